# baseline (device time: 20932 ns/iter reference)
import functools

import jax
import jax.numpy as jnp
from jax import lax
from jax.experimental import pallas as pl
from jax.experimental.pallas import tpu as pltpu

N_DEV = 32

_sem_signal = getattr(pl, "semaphore_signal", None) or pltpu.semaphore_signal
_sem_wait = getattr(pl, "semaphore_wait", None) or pltpu.semaphore_wait
_CompilerParams = getattr(pltpu, "CompilerParams", None) or pltpu.TPUCompilerParams
_DeviceIdType = getattr(pl, "DeviceIdType", None) or pltpu.DeviceIdType

_SEND_ORDER = sorted(range(1, N_DEV), key=lambda d: -min(d, N_DEV - d))
_RECV_ORDER = sorted(range(1, N_DEV), key=lambda d: min(d, N_DEV - d))


def kernel(x):
    m_per, n = x.shape

    def body(x_hbm, out_ref, xv_ref, comm_ref, send_sems, recv_sems, load_sem):
        my_pos = lax.axis_index("i")

        load = pltpu.make_async_copy(x_hbm, xv_ref, load_sem)
        load.start()

        barrier_sem = pltpu.get_barrier_semaphore()
        for d in range(1, N_DEV):
            _sem_signal(
                barrier_sem,
                inc=1,
                device_id=((my_pos + d) % N_DEV,),
                device_id_type=_DeviceIdType.MESH,
            )

        load.wait()
        comm_ref[pl.ds(my_pos, 1), :] = jnp.max(
            xv_ref[:, :], axis=0, keepdims=True
        )
        _sem_wait(barrier_sem, N_DEV - 1)

        sends = []
        for d in _SEND_ORDER:
            s = pltpu.make_async_remote_copy(
                src_ref=comm_ref.at[my_pos],
                dst_ref=comm_ref.at[my_pos],
                send_sem=send_sems.at[d],
                recv_sem=recv_sems.at[my_pos],
                device_id=((my_pos + d) % N_DEV,),
                device_id_type=_DeviceIdType.MESH,
            )
            s.start()
            sends.append(s)

        for d in _RECV_ORDER:
            src_pos = (my_pos + d) % N_DEV
            recv = pltpu.make_async_remote_copy(
                src_ref=comm_ref.at[src_pos],
                dst_ref=comm_ref.at[src_pos],
                send_sem=send_sems.at[d],
                recv_sem=recv_sems.at[src_pos],
                device_id=(my_pos,),
                device_id_type=_DeviceIdType.MESH,
            )
            recv.wait_recv()

        @functools.partial(
            pl.run_scoped, second_barrier=pltpu.SemaphoreType.REGULAR
        )
        def _(second_barrier):
            for d in range(1, N_DEV):
                _sem_signal(
                    second_barrier,
                    inc=1,
                    device_id=((my_pos + d) % N_DEV,),
                    device_id_type=_DeviceIdType.MESH,
                )
            out_ref[:, :] = jnp.max(comm_ref[:, :], axis=0, keepdims=True)
            for s in sends:
                s.wait_send()
            _sem_wait(second_barrier, N_DEV - 1)

    return pl.pallas_call(
        body,
        out_shape=jax.ShapeDtypeStruct((1, n), x.dtype),
        in_specs=[pl.BlockSpec(memory_space=pl.ANY)],
        out_specs=pl.BlockSpec(memory_space=pltpu.VMEM),
        scratch_shapes=[
            pltpu.VMEM((m_per, n), x.dtype),
            pltpu.VMEM((N_DEV, n), x.dtype),
            pltpu.SemaphoreType.DMA((N_DEV,)),
            pltpu.SemaphoreType.DMA((N_DEV,)),
            pltpu.SemaphoreType.DMA,
        ],
        compiler_params=_CompilerParams(collective_id=0),
    )(x)


# device time: 15113 ns/iter; 1.3850x vs baseline; 1.3850x over previous
import functools

import jax
import jax.numpy as jnp
from jax import lax
from jax.experimental import pallas as pl
from jax.experimental.pallas import tpu as pltpu

N_DEV = 32

_sem_signal = getattr(pl, "semaphore_signal", None) or pltpu.semaphore_signal
_sem_wait = getattr(pl, "semaphore_wait", None) or pltpu.semaphore_wait
_CompilerParams = getattr(pltpu, "CompilerParams", None) or pltpu.TPUCompilerParams
_DeviceIdType = getattr(pl, "DeviceIdType", None) or pltpu.DeviceIdType

_SEND_ORDER = sorted(range(1, N_DEV), key=lambda d: -min(d, N_DEV - d))
_RECV_ORDER = sorted(range(1, N_DEV), key=lambda d: min(d, N_DEV - d))


def kernel(x):
    m_per, n = x.shape

    def body(x_hbm, out_ref, xv_ref, comm_ref, send_sems, recv_sems, load_sem):
        my_pos = lax.axis_index("i")

        load = pltpu.make_async_copy(x_hbm, xv_ref, load_sem)
        load.start()

        barrier_sem = pltpu.get_barrier_semaphore()
        for d in range(1, N_DEV):
            _sem_signal(
                barrier_sem,
                inc=1,
                device_id=((my_pos + d) % N_DEV,),
                device_id_type=_DeviceIdType.MESH,
            )

        load.wait()
        comm_ref[pl.ds(my_pos, 1), :] = jnp.max(
            xv_ref[:, :], axis=0, keepdims=True
        )
        _sem_wait(barrier_sem, N_DEV - 1)

        sends = []
        for d in _SEND_ORDER:
            s = pltpu.make_async_remote_copy(
                src_ref=comm_ref.at[my_pos],
                dst_ref=comm_ref.at[my_pos],
                send_sem=send_sems.at[d],
                recv_sem=recv_sems.at[my_pos],
                device_id=((my_pos + d) % N_DEV,),
                device_id_type=_DeviceIdType.MESH,
            )
            s.start()
            sends.append(s)

        for d in _RECV_ORDER:
            src_pos = (my_pos + d) % N_DEV
            recv = pltpu.make_async_remote_copy(
                src_ref=comm_ref.at[src_pos],
                dst_ref=comm_ref.at[src_pos],
                send_sem=send_sems.at[d],
                recv_sem=recv_sems.at[src_pos],
                device_id=(my_pos,),
                device_id_type=_DeviceIdType.MESH,
            )
            recv.wait_recv()

        out_ref[:, :] = jnp.max(comm_ref[:, :], axis=0, keepdims=True)
        for s in sends:
            s.wait_send()

    return pl.pallas_call(
        body,
        out_shape=jax.ShapeDtypeStruct((1, n), x.dtype),
        in_specs=[pl.BlockSpec(memory_space=pl.ANY)],
        out_specs=pl.BlockSpec(memory_space=pltpu.VMEM),
        scratch_shapes=[
            pltpu.VMEM((m_per, n), x.dtype),
            pltpu.VMEM((N_DEV, n), x.dtype),
            pltpu.SemaphoreType.DMA((N_DEV,)),
            pltpu.SemaphoreType.DMA((N_DEV,)),
            pltpu.SemaphoreType.DMA,
        ],
        compiler_params=_CompilerParams(collective_id=0),
    )(x)


# device time: 14397 ns/iter; 1.4539x vs baseline; 1.0497x over previous
import jax
import jax.numpy as jnp
from jax import lax
from jax.experimental import pallas as pl
from jax.experimental.pallas import tpu as pltpu

N_DEV = 32

_sem_signal = getattr(pl, "semaphore_signal", None) or pltpu.semaphore_signal
_sem_wait = getattr(pl, "semaphore_wait", None) or pltpu.semaphore_wait
_CompilerParams = getattr(pltpu, "CompilerParams", None) or pltpu.TPUCompilerParams
_DeviceIdType = getattr(pl, "DeviceIdType", None) or pltpu.DeviceIdType

_SEND_ORDER = sorted(range(1, N_DEV), key=lambda d: -min(d, N_DEV - d))
_RECV_ORDER = sorted(range(1, N_DEV), key=lambda d: min(d, N_DEV - d))


def kernel(x):
    m_per, n = x.shape

    def body(x_hbm, out_ref, xv_ref, comm_ref, send_sems, recv_sems, load_sem):
        my_pos = lax.axis_index("i")

        load = pltpu.make_async_copy(x_hbm, xv_ref, load_sem)
        load.start()

        barrier_sem = pltpu.get_barrier_semaphore()
        _sem_signal(barrier_sem, inc=1)
        _sem_wait(barrier_sem, 1)

        load.wait()
        comm_ref[pl.ds(my_pos, 1), :] = jnp.max(
            xv_ref[:, :], axis=0, keepdims=True
        )

        sends = []
        for d in _SEND_ORDER:
            s = pltpu.make_async_remote_copy(
                src_ref=comm_ref.at[my_pos],
                dst_ref=comm_ref.at[my_pos],
                send_sem=send_sems.at[d],
                recv_sem=recv_sems.at[my_pos],
                device_id=((my_pos + d) % N_DEV,),
                device_id_type=_DeviceIdType.MESH,
            )
            s.start()
            sends.append(s)

        for d in _RECV_ORDER:
            src_pos = (my_pos + d) % N_DEV
            recv = pltpu.make_async_remote_copy(
                src_ref=comm_ref.at[src_pos],
                dst_ref=comm_ref.at[src_pos],
                send_sem=send_sems.at[d],
                recv_sem=recv_sems.at[src_pos],
                device_id=(my_pos,),
                device_id_type=_DeviceIdType.MESH,
            )
            recv.wait_recv()

        out_ref[:, :] = jnp.max(comm_ref[:, :], axis=0, keepdims=True)
        for s in sends:
            s.wait_send()

    return pl.pallas_call(
        body,
        out_shape=jax.ShapeDtypeStruct((1, n), x.dtype),
        in_specs=[pl.BlockSpec(memory_space=pl.ANY)],
        out_specs=pl.BlockSpec(memory_space=pltpu.VMEM),
        scratch_shapes=[
            pltpu.VMEM((m_per, n), x.dtype),
            pltpu.VMEM((N_DEV, n), x.dtype),
            pltpu.SemaphoreType.DMA((N_DEV,)),
            pltpu.SemaphoreType.DMA((N_DEV,)),
            pltpu.SemaphoreType.DMA,
        ],
        compiler_params=_CompilerParams(collective_id=0),
    )(x)
